# Initial kernel scaffold; baseline (speedup 1.0000x reference)
import jax
import jax.numpy as jnp
from jax.experimental import pallas as pl


def kernel(x, edge_index, W0, b0, al0, abl0, ar0, abr0, W1, b1, al1, abl1, ar1, abr1, Wr1, br1, W2, b2, al2, abl2, ar2, abr2, Wr2, br2):
    raise NotImplementedError("write your pallas kernel here")


# speedup vs baseline: 3.7595x; 3.7595x over previous
import jax
import jax.numpy as jnp
from jax.experimental import pallas as pl

_N = 10000
_E = 320000
_D = 128
_HID = 64
_NH = 4
_C = 40


def _edge_phase_tc(a_src, a_dst):
    shape = a_src.shape
    n = a_src.size
    rows = n // 128
    a_src = a_src.reshape(rows, 128)
    a_dst = a_dst.reshape(rows, 128)

    def body(s_ref, d_ref, o_ref):
        v = s_ref[...] + d_ref[...]
        v = jnp.where(v > 0, v, 0.01 * v)
        o_ref[...] = jnp.exp(v)

    blk = 1000 if rows % 1000 == 0 else rows
    out = pl.pallas_call(
        body,
        grid=(rows // blk,),
        in_specs=[pl.BlockSpec((blk, 128), lambda i: (i, 0))] * 2,
        out_specs=pl.BlockSpec((blk, 128), lambda i: (i, 0)),
        out_shape=jax.ShapeDtypeStruct((rows, 128), a_src.dtype),
    )(a_src, a_dst)
    return out.reshape(shape)


def _gat_layer(h, src, dst, W, b, al, abl, ar, abr, Wres, bres):
    nh, din, hid = W.shape
    n = h.shape[0]
    Wf = W.transpose(1, 0, 2).reshape(din, nh * hid)
    ft2 = h @ Wf + b.reshape(1, nh * hid)
    alf = al[:, :, 0].reshape(1, nh * hid)
    arf = ar[:, :, 0].reshape(1, nh * hid)
    a1 = (ft2 * alf).reshape(n, nh, hid).sum(-1) + abl[:, 0][None]
    a2 = (ft2 * arf).reshape(n, nh, hid).sum(-1) + abr[:, 0][None]
    e = _edge_phase_tc(a2[src], a1[dst])
    asum = jax.ops.segment_sum(e, dst, num_segments=n)
    ew = jnp.repeat(e, hid, axis=1)
    vals = ft2[src] * ew
    accum = jax.ops.segment_sum(vals, dst, num_segments=n)
    asum = jnp.where(asum == 0, 1.0, asum)
    accum = accum / jnp.repeat(asum, hid, axis=1)
    if Wres is not None:
        Wrf = Wres.transpose(1, 0, 2).reshape(din, nh * hid)
        accum = accum + (h @ Wrf + bres.reshape(1, nh * hid))
    return jax.nn.elu(accum)


def kernel(x, edge_index, W0, b0, al0, abl0, ar0, abr0, W1, b1, al1, abl1, ar1, abr1, Wr1, br1, W2, b2, al2, abl2, ar2, abr2, Wr2, br2):
    src = edge_index[0]
    dst = edge_index[1]
    h0 = _gat_layer(x, src, dst, W0, b0, al0, abl0, ar0, abr0, None, None)
    h1 = _gat_layer(h0, src, dst, W1, b1, al1, abl1, ar1, abr1, Wr1, br1)
    out = _gat_layer(h1, src, dst, W2[None], b2[None], al2[None], abl2[None],
                     ar2[None], abr2[None], Wr2[None], br2[None])
    return out


# speedup vs baseline: 26.1001x; 6.9425x over previous
import functools

import jax
import jax.numpy as jnp
from jax import lax
from jax.experimental import pallas as pl
from jax.experimental.pallas import tpu as pltpu
from jax.experimental.pallas import tpu_sc as plsc

_N = 10000
_E = 320000
_D = 128
_HID = 64
_NH = 4
_C = 40

_NPAD = 10240
_CH = 128
_NCHUNK = _E // _CH


def _iota16():
    return lax.iota(jnp.int32, 16)


def _sc_kernel_4h():
    mesh = plsc.VectorSubcoreMesh(core_axis_name="c", subcore_axis_name="s")

    @functools.partial(
        pl.kernel,
        out_type=[
            jax.ShapeDtypeStruct((4, _NPAD, 64), jnp.float32),
            jax.ShapeDtypeStruct((2, 16, 2 * _NPAD), jnp.float32),
        ],
        mesh=mesh,
        compiler_params=pltpu.CompilerParams(needs_layout_passes=False, use_tc_tiling_on_sc=False),
        scratch_types=[
            pltpu.VMEM((4 * _NPAD,), jnp.float32),
            pltpu.VMEM((1, _CH), jnp.int32),
            pltpu.VMEM((1, _CH), jnp.int32),
            pltpu.VMEM((_CH, 64), jnp.float32),
            pltpu.VMEM((1, _CH), jnp.float32),
            pltpu.VMEM((2 * _NPAD,), jnp.float32),
            pltpu.VMEM((16, 64), jnp.float32),
            pltpu.VMEM_SHARED((_NPAD, 64), jnp.float32),
            pltpu.SemaphoreType.DMA,
        ],
    )
    def k(ei, a12, ft, accum_out, asum_out,
          a12_v, srcb, dstb, rows_v, e_v, asum_part, zero_v, accum_sh, sem):
        c = lax.axis_index("c")
        s = lax.axis_index("s")
        zv = jnp.zeros((16,), jnp.float32)

        pltpu.sync_copy(a12.at[c], a12_v)

        for i in range(16):
            for j in range(4):
                zero_v[i, pl.ds(j * 16, 16)] = zv

        def zpart(t, _):
            asum_part[pl.ds(t * 16, 16)] = zv
            return 0
        lax.fori_loop(0, 2 * _NPAD // 16, zpart, 0)

        nloc = 156 + jnp.where(s < 4, 1, 0)

        for h in range(2):
            def zacc(t, _):
                pltpu.sync_copy(zero_v,
                                accum_sh.at[pl.ds(s * 640 + t * 16, 16)])
                return 0
            lax.fori_loop(0, 40, zacc, 0)
            plsc.subcore_barrier()

            def chunk_body(j, _):
                ch = j * 16 + s
                base = ch * _CH
                pltpu.sync_copy(ei.at[0, pl.ds(base, _CH)], srcb.at[0])
                pltpu.sync_copy(ei.at[1, pl.ds(base, _CH)], dstb.at[0])
                pltpu.async_copy(ft.at[2 * c + h].at[srcb.at[0]],
                                 rows_v, sem).wait()

                for g in range(8):
                    sidx = srcb[0, pl.ds(g * 16, 16)]
                    didx = dstb[0, pl.ds(g * 16, 16)]
                    va1 = plsc.load_gather(a12_v, [didx + h * _NPAD])
                    va2 = plsc.load_gather(
                        a12_v, [sidx + (2 * _NPAD + h * _NPAD)])
                    v = va1 + va2
                    e = jnp.exp(jnp.where(v > 0, v, 0.01 * v))
                    e_v[0, pl.ds(g * 16, 16)] = e
                    plsc.addupdate_scatter(asum_part, [didx + h * _NPAD], e)

                for g in range(8):
                    ev0 = e_v[0, pl.ds(g * 16, 16)]
                    for l in range(16):
                        i = g * 16 + l
                        e0 = ev0[l]
                        for kk in range(4):
                            rows_v[i, pl.ds(kk * 16, 16)] = (
                                rows_v[i, pl.ds(kk * 16, 16)] * e0)

                pltpu.sync_copy(rows_v, accum_sh.at[dstb.at[0]], add=True)
                return 0

            lax.fori_loop(0, nloc, chunk_body, 0)
            plsc.subcore_barrier()
            pltpu.sync_copy(accum_sh.at[pl.ds(s * 640, 640)],
                            accum_out.at[2 * c + h, pl.ds(s * 640, 640)])

        pltpu.sync_copy(asum_part, asum_out.at[c, s])

    return k


def _sc_kernel_1h():
    mesh = plsc.VectorSubcoreMesh(core_axis_name="c", subcore_axis_name="s")

    @functools.partial(
        pl.kernel,
        out_type=[
            jax.ShapeDtypeStruct((2, _NPAD, 64), jnp.float32),
            jax.ShapeDtypeStruct((2, 16, _NPAD), jnp.float32),
        ],
        mesh=mesh,
        compiler_params=pltpu.CompilerParams(needs_layout_passes=False, use_tc_tiling_on_sc=False),
        scratch_types=[
            pltpu.VMEM((2 * _NPAD,), jnp.float32),
            pltpu.VMEM((1, _CH), jnp.int32),
            pltpu.VMEM((1, _CH), jnp.int32),
            pltpu.VMEM((_CH, 64), jnp.float32),
            pltpu.VMEM((1, _CH), jnp.float32),
            pltpu.VMEM((_NPAD,), jnp.float32),
            pltpu.VMEM((16, 64), jnp.float32),
            pltpu.VMEM_SHARED((_NPAD, 64), jnp.float32),
            pltpu.SemaphoreType.DMA,
        ],
    )
    def k(ei, a12, ft, accum_out, asum_out,
          a12_v, srcb, dstb, rows_v, e_v, asum_part, zero_v, accum_sh, sem):
        c = lax.axis_index("c")
        s = lax.axis_index("s")
        zv = jnp.zeros((16,), jnp.float32)

        pltpu.sync_copy(a12, a12_v)

        for i in range(16):
            for j in range(4):
                zero_v[i, pl.ds(j * 16, 16)] = zv

        def zacc(t, _):
            pltpu.sync_copy(zero_v, accum_sh.at[pl.ds(s * 640 + t * 16, 16)])
            return 0
        lax.fori_loop(0, 40, zacc, 0)

        def zpart(t, _):
            asum_part[pl.ds(t * 16, 16)] = zv
            return 0
        lax.fori_loop(0, _NPAD // 16, zpart, 0)

        plsc.subcore_barrier()

        nloc = 78 + jnp.where(s < 2, 1, 0)

        def chunk_body(j, _):
            ch = c * (_NCHUNK // 2) + j * 16 + s
            base = ch * _CH
            pltpu.sync_copy(ei.at[0, pl.ds(base, _CH)], srcb.at[0])
            pltpu.sync_copy(ei.at[1, pl.ds(base, _CH)], dstb.at[0])
            pltpu.async_copy(ft.at[srcb.at[0]], rows_v, sem).wait()

            for g in range(8):
                sidx = srcb[0, pl.ds(g * 16, 16)]
                didx = dstb[0, pl.ds(g * 16, 16)]
                va1 = plsc.load_gather(a12_v, [didx])
                va2 = plsc.load_gather(a12_v, [sidx + _NPAD])
                v = va1 + va2
                e = jnp.exp(jnp.where(v > 0, v, 0.01 * v))
                e_v[0, pl.ds(g * 16, 16)] = e
                plsc.addupdate_scatter(asum_part, [didx], e)

            for g in range(8):
                ev0 = e_v[0, pl.ds(g * 16, 16)]
                for l in range(16):
                    i = g * 16 + l
                    e0 = ev0[l]
                    for kk in range(4):
                        rows_v[i, pl.ds(kk * 16, 16)] = (
                            rows_v[i, pl.ds(kk * 16, 16)] * e0)

            pltpu.sync_copy(rows_v, accum_sh.at[dstb.at[0]], add=True)
            return 0

        lax.fori_loop(0, nloc, chunk_body, 0)

        pltpu.sync_copy(asum_part, asum_out.at[c, s])

        plsc.subcore_barrier()
        pltpu.sync_copy(accum_sh.at[pl.ds(s * 640, 640)],
                        accum_out.at[c, pl.ds(s * 640, 640)])

    return k


def _pad_rows(a, npad):
    return jnp.pad(a, ((0, npad - a.shape[0]),) + ((0, 0),) * (a.ndim - 1))


def _layer_4h(h, ei, W, b, al, abl, ar, abr, Wres, bres, sck):
    nh, din, hid = W.shape
    Wf = W.transpose(1, 0, 2).reshape(din, nh * hid)
    ft2 = h @ Wf + b.reshape(1, nh * hid)
    alf = al[:, :, 0].reshape(1, nh * hid)
    arf = ar[:, :, 0].reshape(1, nh * hid)
    a1 = (ft2 * alf).reshape(_N, nh, hid).sum(-1) + abl[:, 0][None]
    a2 = (ft2 * arf).reshape(_N, nh, hid).sum(-1) + abr[:, 0][None]

    a1p = _pad_rows(a1, _NPAD).T.reshape(2, 2, _NPAD)
    a2p = _pad_rows(a2, _NPAD).T.reshape(2, 2, _NPAD)
    a12 = jnp.stack([a1p, a2p], axis=1).reshape(2, 4 * _NPAD)
    ftp = _pad_rows(ft2, _NPAD).reshape(_NPAD, 4, 64).transpose(1, 0, 2)

    accum, asum_parts = sck(ei, a12, ftp)
    accum = accum.transpose(1, 0, 2).reshape(_NPAD, 256)[:_N]
    asum = asum_parts.sum(axis=1).reshape(4, _NPAD)[:, :_N].T
    asum = jnp.where(asum == 0, 1.0, asum)
    out = accum / jnp.repeat(asum, hid, axis=1)
    if Wres is not None:
        Wrf = Wres.transpose(1, 0, 2).reshape(din, nh * hid)
        out = out + (h @ Wrf + bres.reshape(1, nh * hid))
    return jax.nn.elu(out)


def _layer_1h(h, ei, W, b, al, abl, ar, abr, Wres, bres, sck):
    din, cc = W.shape
    ft2 = h @ W + b[None]
    a1 = ft2 @ al[:, 0] + abl[0]
    a2 = ft2 @ ar[:, 0] + abr[0]
    a12 = jnp.concatenate([jnp.pad(a1, (0, _NPAD - _N)),
                           jnp.pad(a2, (0, _NPAD - _N))])
    ftp = _pad_rows(jnp.pad(ft2, ((0, 0), (0, 64 - cc))), _NPAD)

    accum, asum_parts = sck(ei, a12, ftp)
    accum = accum.sum(axis=0)[:_N, :cc]
    asum = asum_parts.sum(axis=(0, 1))[:_N]
    asum = jnp.where(asum == 0, 1.0, asum)
    out = accum / asum[:, None]
    out = out + (h @ Wres + bres[None])
    return jax.nn.elu(out)


def kernel(x, edge_index, W0, b0, al0, abl0, ar0, abr0, W1, b1, al1, abl1, ar1, abr1, Wr1, br1, W2, b2, al2, abl2, ar2, abr2, Wr2, br2):
    ei = edge_index
    sck4 = _sc_kernel_4h()
    sck1 = _sc_kernel_1h()
    h0 = _layer_4h(x, ei, W0, b0, al0, abl0, ar0, abr0, None, None, sck4)
    h1 = _layer_4h(h0, ei, W1, b1, al1, abl1, ar1, abr1, Wr1, br1, sck4)
    out = _layer_1h(h1, ei, W2, b2, al2, abl2, ar2, abr2, Wr2, br2, sck1)
    return out
